# Initial kernel scaffold; baseline (speedup 1.0000x reference)
import jax
import jax.numpy as jnp
from jax.experimental import pallas as pl


def kernel(x, adj):
    raise NotImplementedError("write your pallas kernel here")


# speedup vs baseline: 6.2794x; 6.2794x over previous
import functools

import jax
import jax.numpy as jnp
from jax.experimental import pallas as pl
from jax.experimental.pallas import tpu as pltpu

N = 4096
D = 64
BI = 128
BJ = 128
NI = N // BI
NJ = N // BJ

NEG = float("-inf")


def _dense_body(adj_ref, x_ref, out_ref, acc_ref):
    j = pl.program_id(1)

    @pl.when(j == 0)
    def _init():
        acc_ref[...] = jnp.full((BI, D), NEG, jnp.float32)

    acc = acc_ref[...]
    adj_blk = adj_ref[...]
    x_blk = x_ref[...]
    for k in range(BJ):
        m = adj_blk[:, k : k + 1] > 0
        cand = jnp.where(m, x_blk[k : k + 1, :], NEG)
        acc = jnp.maximum(acc, cand)
    acc_ref[...] = acc

    @pl.when(j == NJ - 1)
    def _fin():
        a = acc_ref[...]
        out_ref[...] = jnp.where(a == NEG, 0.0, a)


@jax.jit
def _dense(x, adj):
    return pl.pallas_call(
        _dense_body,
        grid=(NI, NJ),
        in_specs=[
            pl.BlockSpec((BI, BJ), lambda i, j: (i, j)),
            pl.BlockSpec((BJ, D), lambda i, j: (j, 0)),
        ],
        out_specs=pl.BlockSpec((BI, D), lambda i, j: (i, 0)),
        out_shape=jax.ShapeDtypeStruct((N, D), jnp.float32),
        scratch_shapes=[pltpu.VMEM((BI, D), jnp.float32)],
        compiler_params=pltpu.CompilerParams(
            dimension_semantics=("arbitrary", "arbitrary"),
        ),
    )(adj, x)


def kernel(x, adj):
    return _dense(x, adj)
